# baseline (speedup 1.0000x reference)
import functools

import jax
import jax.numpy as jnp
from jax import lax
from jax.experimental import pallas as pl
from jax.experimental.pallas import tpu as pltpu
from jax.experimental.pallas import tpu_sc as plsc

_N_ATOMS = 1048576
_N_TYPES = 100
_N_STRUCT = 16384
_LANES = 16
_NUM_CORES = 2
_NUM_SUBCORES = 16
_NW = _NUM_CORES * _NUM_SUBCORES
_CHUNK = _N_ATOMS // _NW
_BLOCKS = _CHUNK // _LANES


def _make_stage1():
    mesh = plsc.VectorSubcoreMesh(core_axis_name="c", subcore_axis_name="s")

    @functools.partial(
        pl.kernel,
        out_type=jax.ShapeDtypeStruct((_NW, _N_STRUCT), jnp.float32),
        mesh=mesh,
        scratch_types=[
            pltpu.VMEM((1, _N_TYPES), jnp.float32),
            pltpu.VMEM((_CHUNK,), jnp.int32),
            pltpu.VMEM((_CHUNK,), jnp.int32),
            pltpu.VMEM((_N_STRUCT,), jnp.float32),
            pltpu.SemaphoreType.DMA,
            pltpu.SemaphoreType.DMA((2,)),
        ],
        compiler_params=pltpu.CompilerParams(needs_layout_passes=False),
    )
    def stage1(w_hbm, t_hbm, s_hbm, part_hbm, w_v, t_v, s_v, d_v, sem, csem):
        wid = lax.axis_index("s") * _NUM_CORES + lax.axis_index("c")
        base = wid * _CHUNK
        half = _CHUNK // 2
        cp_w = pltpu.async_copy(w_hbm, w_v, sem)
        cp_t0 = pltpu.async_copy(
            t_hbm.at[pl.ds(base, half)], t_v.at[pl.ds(0, half)], csem.at[0])
        cp_s0 = pltpu.async_copy(
            s_hbm.at[pl.ds(base, half)], s_v.at[pl.ds(0, half)], csem.at[0])
        cp_t1 = pltpu.async_copy(
            t_hbm.at[pl.ds(base + half, half)], t_v.at[pl.ds(half, half)],
            csem.at[1])
        cp_s1 = pltpu.async_copy(
            s_hbm.at[pl.ds(base + half, half)], s_v.at[pl.ds(half, half)],
            csem.at[1])

        zeros = jnp.zeros((_LANES,), jnp.float32)

        with jax.named_scope("zero_acc"):
            @pl.loop(0, _N_STRUCT // _LANES, unroll=8)
            def _zero(i):
                d_v[pl.ds(i * _LANES, _LANES)] = zeros

        with jax.named_scope("dma_wait"):
            cp_w.wait()
            cp_t0.wait()
            cp_s0.wait()

        iota = lax.iota(jnp.int32, _LANES)
        shift_idx = jnp.minimum(iota + 1, _LANES - 1)
        last = iota == (_LANES - 1)
        zero_i = jnp.zeros((_LANES,), jnp.int32)
        dnums = lax.GatherDimensionNumbers(
            offset_dims=(), collapsed_slice_dims=(0,), start_index_map=(0,))

        def _shift_up(v):
            return lax.gather(
                v, shift_idx[:, None], dnums, slice_sizes=(1,),
                mode=lax.GatherScatterMode.PROMISE_IN_BOUNDS)

        def _block(b):
            off = pl.multiple_of(b * _LANES, _LANES)
            t = t_v[pl.ds(off, _LANES)]
            s = s_v[pl.ds(off, _LANES)]
            w = plsc.load_gather(w_v, [zero_i, t])
            c = plsc.cumsum(w)
            s_next = _shift_up(s)
            run_end = s != s_next
            m_end = run_end | last
            plsc.addupdate_scatter(d_v, [s], c, mask=m_end)
            plsc.addupdate_scatter(d_v, [s_next], -c, mask=run_end)

        with jax.named_scope("main0"):
            plsc.parallel_loop(0, _BLOCKS // 2, unroll=8)(_block)

        with jax.named_scope("dma_wait1"):
            cp_t1.wait()
            cp_s1.wait()

        with jax.named_scope("main1"):
            plsc.parallel_loop(_BLOCKS // 2, _BLOCKS, unroll=8)(_block)

        with jax.named_scope("out_dma"):
            pltpu.sync_copy(d_v, part_hbm.at[wid])

    return stage1


def _reduce_body(p_ref, o_ref):
    o_ref[...] = jnp.sum(p_ref[...], axis=0)


_RED_BLK = 2048

_stage2 = pl.pallas_call(
    _reduce_body,
    grid=(_N_STRUCT // _RED_BLK,),
    in_specs=[pl.BlockSpec((_NW, _RED_BLK), lambda i: (0, i))],
    out_specs=pl.BlockSpec((_RED_BLK,), lambda i: (i,)),
    out_shape=jax.ShapeDtypeStruct((_N_STRUCT,), jnp.float32),
)

_stage1 = _make_stage1()


@jax.jit
def _impl(weights, types, segment_ids):
    parts = _stage1(weights.astype(jnp.float32), types.astype(jnp.int32),
                    segment_ids.astype(jnp.int32))
    return _stage2(parts).reshape(_N_STRUCT, 1)


def kernel(weights, types, segment_ids):
    return _impl(weights, types, segment_ids)


# speedup vs baseline: 1.1308x; 1.1308x over previous
import functools

import jax
import jax.numpy as jnp
from jax import lax
from jax.experimental import pallas as pl
from jax.experimental.pallas import tpu as pltpu
from jax.experimental.pallas import tpu_sc as plsc

_N_ATOMS = 1048576
_N_TYPES = 100
_N_STRUCT = 16384
_LANES = 16
_NUM_CORES = 2
_NUM_SUBCORES = 16
_NW = _NUM_CORES * _NUM_SUBCORES
_CHUNK = _N_ATOMS // _NW
_BLOCKS = _CHUNK // _LANES
_PIPE = 4


def _make_stage1():
    mesh = plsc.VectorSubcoreMesh(core_axis_name="c", subcore_axis_name="s")

    @functools.partial(
        pl.kernel,
        out_type=jax.ShapeDtypeStruct((_NW, _N_STRUCT), jnp.float32),
        mesh=mesh,
        scratch_types=[
            pltpu.VMEM((1, _N_TYPES), jnp.float32),
            pltpu.VMEM((_CHUNK,), jnp.int32),
            pltpu.VMEM((_CHUNK,), jnp.int32),
            pltpu.VMEM((_N_STRUCT,), jnp.float32),
            pltpu.SemaphoreType.DMA,
            pltpu.SemaphoreType.DMA((_PIPE,)),
        ],
        compiler_params=pltpu.CompilerParams(needs_layout_passes=False),
    )
    def stage1(w_hbm, t_hbm, s_hbm, part_hbm, w_v, t_v, s_v, d_v, sem, csem):
        wid = lax.axis_index("s") * _NUM_CORES + lax.axis_index("c")
        base = wid * _CHUNK
        sub = _CHUNK // _PIPE

        def _issue(j):
            tj = pltpu.async_copy(
                t_hbm.at[pl.ds(base + j * sub, sub)],
                t_v.at[pl.ds(j * sub, sub)], csem.at[j])
            sj = pltpu.async_copy(
                s_hbm.at[pl.ds(base + j * sub, sub)],
                s_v.at[pl.ds(j * sub, sub)], csem.at[j])
            return tj, sj

        cp_w = pltpu.async_copy(w_hbm, w_v, sem)
        cp0 = _issue(0)

        zeros = jnp.zeros((_LANES,), jnp.float32)

        with jax.named_scope("zero_acc"):
            @pl.loop(0, _N_STRUCT // _LANES, unroll=8)
            def _zero(i):
                d_v[pl.ds(i * _LANES, _LANES)] = zeros

        with jax.named_scope("dma_wait"):
            cp_w.wait()

        iota = lax.iota(jnp.int32, _LANES)
        shift_idx = jnp.minimum(iota + 1, _LANES - 1)
        last = iota == (_LANES - 1)
        zero_i = jnp.zeros((_LANES,), jnp.int32)
        dnums = lax.GatherDimensionNumbers(
            offset_dims=(), collapsed_slice_dims=(0,), start_index_map=(0,))

        def _shift_up(v):
            return lax.gather(
                v, shift_idx[:, None], dnums, slice_sizes=(1,),
                mode=lax.GatherScatterMode.PROMISE_IN_BOUNDS)

        def _block(b):
            off = pl.multiple_of(b * _LANES, _LANES)
            t = t_v[pl.ds(off, _LANES)]
            s = s_v[pl.ds(off, _LANES)]
            w = plsc.load_gather(w_v, [zero_i, t])
            c = plsc.cumsum(w)
            s_next = _shift_up(s)
            run_end = s != s_next
            m_end = run_end | last
            plsc.addupdate_scatter(d_v, [s], c, mask=m_end)
            plsc.addupdate_scatter(d_v, [s_next], -c, mask=run_end)

        sub_blocks = _BLOCKS // _PIPE
        cp = cp0
        for j in range(_PIPE):
            with jax.named_scope(f"wait{j}"):
                cp[0].wait()
                cp[1].wait()
            if j + 1 < _PIPE:
                cp = _issue(j + 1)
            with jax.named_scope(f"main{j}"):
                plsc.parallel_loop(j * sub_blocks, (j + 1) * sub_blocks,
                                   unroll=8)(_block)

        with jax.named_scope("out_dma"):
            pltpu.sync_copy(d_v, part_hbm.at[wid])

    return stage1


def _reduce_body(p_ref, o_ref):
    o_ref[...] = jnp.sum(p_ref[...], axis=0)


_stage2 = pl.pallas_call(
    _reduce_body,
    out_shape=jax.ShapeDtypeStruct((_N_STRUCT,), jnp.float32),
)

_stage1 = _make_stage1()


@jax.jit
def _impl(weights, types, segment_ids):
    parts = _stage1(weights.astype(jnp.float32), types.astype(jnp.int32),
                    segment_ids.astype(jnp.int32))
    return _stage2(parts).reshape(_N_STRUCT, 1)


def kernel(weights, types, segment_ids):
    return _impl(weights, types, segment_ids)
